# baseline (speedup 1.0000x reference)
import functools

import jax
import jax.numpy as jnp
import numpy as np
from jax import lax
from jax.experimental import pallas as pl
from jax.experimental.pallas import tpu as pltpu
from jax.experimental.pallas import tpu_sc as plsc

N_POINTS = 4_000_000
N_L, N_B, N_MU = 90, 20, 16
N_BINS = N_L * N_B * N_MU

_GRID_LO = np.array([-90.0, -12.0, 10.0], dtype=np.float32)
_GRID_HI = np.array([90.0, 12.0, 13.0], dtype=np.float32)
_DX = ((_GRID_HI - _GRID_LO) / np.array([N_L, N_B, N_MU], dtype=np.float32)).astype(np.float32)

NUM_WORKERS = 32
PER_TILE = N_POINTS // NUM_WORKERS
CHUNK = 5000
N_CHUNKS = PER_TILE // CHUNK
VECS = (CHUNK + 15) // 16

_mesh = plsc.VectorSubcoreMesh(core_axis_name="c", subcore_axis_name="s")


@functools.partial(
    pl.kernel,
    out_type=jax.ShapeDtypeStruct((NUM_WORKERS, N_BINS), jnp.float32),
    mesh=_mesh,
    compiler_params=pltpu.CompilerParams(needs_layout_passes=False),
    scratch_types=[
        pltpu.VMEM((CHUNK * 3 + 40,), jnp.float32),
        pltpu.VMEM((CHUNK + 16,), jnp.float32),
        pltpu.VMEM((N_BINS,), jnp.float32),
    ],
)
def _sc_hist(pts_hbm, mass_hbm, out_hbm, pbuf, mbuf, hist):
    cid = lax.axis_index("c")
    sid = lax.axis_index("s")
    wid = sid * 2 + cid
    base_pt = wid * PER_TILE

    zeros = jnp.zeros((16,), jnp.float32)
    def _zero(i, _):
        hist[pl.ds(i * 16, 16)] = zeros
        return ()
    lax.fori_loop(0, N_BINS // 16, _zero, ())
    pbuf[pl.ds(CHUNK * 3, 16)] = zeros
    pbuf[pl.ds(CHUNK * 3 + 16, 16)] = zeros
    mbuf[pl.ds(CHUNK, 16)] = zeros

    lane = lax.iota(jnp.int32, 16)
    lane3 = lane * 3
    lo0 = jnp.float32(_GRID_LO[0]); dx0 = jnp.float32(_DX[0])
    lo1 = jnp.float32(_GRID_LO[1]); dx1 = jnp.float32(_DX[1])
    lo2 = jnp.float32(_GRID_LO[2]); dx2 = jnp.float32(_DX[2])

    def _chunk(ci, _):
        pt_off = (base_pt + ci * CHUNK) * 3
        m_off = base_pt + ci * CHUNK
        pltpu.sync_copy(pts_hbm.at[pl.ds(pt_off, CHUNK * 3)], pbuf.at[pl.ds(0, CHUNK * 3)])
        pltpu.sync_copy(mass_hbm.at[pl.ds(m_off, CHUNK)], mbuf.at[pl.ds(0, CHUNK)])

        def _vec(j, _):
            b = j * 48 + lane3
            lv = plsc.load_gather(pbuf, [b])
            bv = plsc.load_gather(pbuf, [b + 1])
            uv = plsc.load_gather(pbuf, [b + 2])
            mv = mbuf[pl.ds(j * 16, 16)]
            i0 = jnp.clip(((lv - lo0) / dx0).astype(jnp.int32), 0, N_L - 1)
            i1 = jnp.clip(((bv - lo1) / dx1).astype(jnp.int32), 0, N_B - 1)
            i2 = jnp.clip(((uv - lo2) / dx2).astype(jnp.int32), 0, N_MU - 1)
            flat = i0 * (N_B * N_MU) + i1 * N_MU + i2
            valid = (j * 16 + lane) < CHUNK
            plsc.addupdate_scatter(hist, [flat], mv, mask=valid)
            return ()

        lax.fori_loop(0, VECS, _vec, ())
        return ()

    lax.fori_loop(0, N_CHUNKS, _chunk, ())
    pltpu.sync_copy(hist, out_hbm.at[wid])


def _tc_body(p_ref, lfr_ref, out_ref):
    h = jnp.sum(p_ref[...], axis=0)
    lfr = lfr_ref[...]
    w = jnp.concatenate([lfr[:, 5 - t:21 - t] for t in range(6)], axis=0)
    out_ref[...] = jax.lax.dot_general(
        h, w, (((1,), (1,)), ((), ())), preferred_element_type=jnp.float32)


_tc_reduce_conv = pl.pallas_call(
    _tc_body,
    out_shape=jax.ShapeDtypeStruct((N_L * N_B, 6), jnp.float32),
    in_specs=[
        pl.BlockSpec((NUM_WORKERS, N_L * N_B, N_MU), lambda: (0, 0, 0)),
        pl.BlockSpec((1, 21), lambda: (0, 0)),
    ],
    out_specs=pl.BlockSpec((N_L * N_B, 6), lambda: (0, 0)),
)


def kernel(l_b_mu, masses, lf_number):
    pts_flat = l_b_mu.reshape(-1)
    partials = _sc_hist(pts_flat, masses)
    lf_rev = lf_number[::-1].reshape(1, 21)
    out = _tc_reduce_conv(partials.reshape(NUM_WORKERS, N_L * N_B, N_MU), lf_rev)
    return out.reshape(N_L, N_B, 6)


# speedup vs baseline: 27.7202x; 27.7202x over previous
import functools

import jax
import jax.numpy as jnp
import numpy as np
from jax import lax
from jax.experimental import pallas as pl
from jax.experimental.pallas import tpu as pltpu
from jax.experimental.pallas import tpu_sc as plsc

N_POINTS = 4_000_000
N_L, N_B, N_MU = 90, 20, 16
N_BINS = N_L * N_B * N_MU

_GRID_LO = np.array([-90.0, -12.0, 10.0], dtype=np.float32)
_GRID_HI = np.array([90.0, 12.0, 13.0], dtype=np.float32)
_DX = ((_GRID_HI - _GRID_LO) / np.array([N_L, N_B, N_MU], dtype=np.float32)).astype(np.float32)

NUM_WORKERS = 32
PER_TILE = N_POINTS // NUM_WORKERS
CHUNK = 5000
N_CHUNKS = PER_TILE // CHUNK
VECS = (CHUNK + 15) // 16

_mesh = plsc.VectorSubcoreMesh(core_axis_name="c", subcore_axis_name="s")


@functools.partial(
    pl.kernel,
    out_type=jax.ShapeDtypeStruct((NUM_WORKERS, N_BINS), jnp.float32),
    mesh=_mesh,
    compiler_params=pltpu.CompilerParams(needs_layout_passes=False),
    scratch_types=[
        pltpu.VMEM((CHUNK + 16,), jnp.float32),
        pltpu.VMEM((CHUNK + 16,), jnp.float32),
        pltpu.VMEM((CHUNK + 16,), jnp.float32),
        pltpu.VMEM((CHUNK + 16,), jnp.float32),
        pltpu.VMEM((N_BINS,), jnp.float32),
    ],
)
def _sc_hist(l_hbm, b_hbm, mu_hbm, mass_hbm, out_hbm, lbuf, bbuf, ubuf, mbuf, hist):
    cid = lax.axis_index("c")
    sid = lax.axis_index("s")
    wid = sid * 2 + cid
    base_pt = wid * PER_TILE

    zeros = jnp.zeros((16,), jnp.float32)
    def _zero(i, _):
        hist[pl.ds(i * 16, 16)] = zeros
        return ()
    lax.fori_loop(0, N_BINS // 16, _zero, ())
    lbuf[pl.ds(CHUNK, 16)] = zeros
    bbuf[pl.ds(CHUNK, 16)] = zeros
    ubuf[pl.ds(CHUNK, 16)] = zeros
    mbuf[pl.ds(CHUNK, 16)] = zeros

    lane = lax.iota(jnp.int32, 16)
    lo0 = jnp.float32(_GRID_LO[0]); dx0 = jnp.float32(_DX[0])
    lo1 = jnp.float32(_GRID_LO[1]); dx1 = jnp.float32(_DX[1])
    lo2 = jnp.float32(_GRID_LO[2]); dx2 = jnp.float32(_DX[2])

    def _chunk(ci, _):
        off = base_pt + ci * CHUNK
        pltpu.sync_copy(l_hbm.at[pl.ds(off, CHUNK)], lbuf.at[pl.ds(0, CHUNK)])
        pltpu.sync_copy(b_hbm.at[pl.ds(off, CHUNK)], bbuf.at[pl.ds(0, CHUNK)])
        pltpu.sync_copy(mu_hbm.at[pl.ds(off, CHUNK)], ubuf.at[pl.ds(0, CHUNK)])
        pltpu.sync_copy(mass_hbm.at[pl.ds(off, CHUNK)], mbuf.at[pl.ds(0, CHUNK)])

        def _vec(j, _):
            s = j * 16
            lv = lbuf[pl.ds(s, 16)]
            bv = bbuf[pl.ds(s, 16)]
            uv = ubuf[pl.ds(s, 16)]
            mv = mbuf[pl.ds(s, 16)]
            i0 = jnp.clip(((lv - lo0) / dx0).astype(jnp.int32), 0, N_L - 1)
            i1 = jnp.clip(((bv - lo1) / dx1).astype(jnp.int32), 0, N_B - 1)
            i2 = jnp.clip(((uv - lo2) / dx2).astype(jnp.int32), 0, N_MU - 1)
            flat = i0 * (N_B * N_MU) + i1 * N_MU + i2
            valid = (s + lane) < CHUNK
            plsc.addupdate_scatter(hist, [flat], mv, mask=valid)
            return ()

        lax.fori_loop(0, VECS, _vec, ())
        return ()

    lax.fori_loop(0, N_CHUNKS, _chunk, ())
    pltpu.sync_copy(hist, out_hbm.at[wid])


def _tc_body(p_ref, lfr_ref, out_ref):
    h = jnp.sum(p_ref[...], axis=0)
    lfr = lfr_ref[...]
    w = jnp.concatenate([lfr[:, 5 - t:21 - t] for t in range(6)], axis=0)
    out_ref[...] = jax.lax.dot_general(
        h, w, (((1,), (1,)), ((), ())), preferred_element_type=jnp.float32)


_tc_reduce_conv = pl.pallas_call(
    _tc_body,
    out_shape=jax.ShapeDtypeStruct((N_L * N_B, 6), jnp.float32),
    in_specs=[
        pl.BlockSpec((NUM_WORKERS, N_L * N_B, N_MU), lambda: (0, 0, 0)),
        pl.BlockSpec((1, 21), lambda: (0, 0)),
    ],
    out_specs=pl.BlockSpec((N_L * N_B, 6), lambda: (0, 0)),
)


def kernel(l_b_mu, masses, lf_number):
    l = l_b_mu[:, 0]
    b = l_b_mu[:, 1]
    mu = l_b_mu[:, 2]
    partials = _sc_hist(l, b, mu, masses)
    lf_rev = lf_number[::-1].reshape(1, 21)
    out = _tc_reduce_conv(partials.reshape(NUM_WORKERS, N_L * N_B, N_MU), lf_rev)
    return out.reshape(N_L, N_B, 6)
